# Initial kernel scaffold; baseline (speedup 1.0000x reference)
import jax
import jax.numpy as jnp
from jax.experimental import pallas as pl


def kernel(x, edge_index, priors, We1, be1, We2, be2, Wg1, bg1, Wg2, bg2, Wg3, bg3, Wp1, bp1, Wp2, bp2):
    raise NotImplementedError("write your pallas kernel here")


# speedup vs baseline: 27.5554x; 27.5554x over previous
import functools

import jax
import jax.numpy as jnp
from jax import lax
from jax.experimental import pallas as pl
from jax.experimental.pallas import tpu as pltpu
from jax.experimental.pallas import tpu_sc as plsc

N = 10000
NPAD = 10240
E = 320000
NW = 32
K = 80
C = 125
STRIPE = NPAD // 16
DEGW = 8
BT = 1024
DIN = 128
DH = 32


def _sc_mesh():
    return plsc.VectorSubcoreMesh(core_axis_name="c", subcore_axis_name="s")


def _sc_agg(g_pad, src3, dst3, z32):
    @functools.partial(
        pl.kernel,
        out_type=jax.ShapeDtypeStruct((2, NPAD, DH), jnp.float32),
        mesh=_sc_mesh(),
        compiler_params=pltpu.CompilerParams(use_tc_tiling_on_sc=False),
        scratch_types=[
            pltpu.VMEM((K, C), jnp.int32),
            pltpu.VMEM((K, C), jnp.int32),
            pltpu.VMEM((C, DH), jnp.float32),
            pltpu.VMEM_SHARED((NPAD, DH), jnp.float32),
            pltpu.SemaphoreType.DMA,
        ],
    )
    def k(g_hbm, src_hbm, dst_hbm, z_hbm, out_hbm, src_v, dst_v, rows_v, acc, sem):
        c = lax.axis_index("c")
        s = lax.axis_index("s")
        wid = c * 16 + s
        pltpu.sync_copy(z_hbm.at[pl.ds(s * STRIPE, STRIPE)],
                        acc.at[pl.ds(s * STRIPE, STRIPE)])
        pltpu.sync_copy(src_hbm.at[wid], src_v)
        pltpu.sync_copy(dst_hbm.at[wid], dst_v)
        plsc.subcore_barrier()

        def body(j, carry):
            pltpu.async_copy(g_hbm.at[src_v.at[j]], rows_v, sem).wait()
            pltpu.sync_copy(rows_v, acc.at[dst_v.at[j]], add=True)
            return carry

        lax.fori_loop(0, K, body, 0)
        plsc.subcore_barrier()
        pltpu.sync_copy(acc.at[pl.ds(s * STRIPE, STRIPE)],
                        out_hbm.at[c, pl.ds(s * STRIPE, STRIPE)])

    return k(g_pad, src3, dst3, z32)


def _sc_deg(dst3, ones_c, z8):
    @functools.partial(
        pl.kernel,
        out_type=jax.ShapeDtypeStruct((2, NPAD, DEGW), jnp.float32),
        mesh=_sc_mesh(),
        compiler_params=pltpu.CompilerParams(use_tc_tiling_on_sc=False),
        scratch_types=[
            pltpu.VMEM((K, C), jnp.int32),
            pltpu.VMEM((C, DEGW), jnp.float32),
            pltpu.VMEM_SHARED((NPAD, DEGW), jnp.float32),
        ],
    )
    def k(dst_hbm, ones_hbm, z_hbm, out_hbm, dst_v, ones_v, acc):
        c = lax.axis_index("c")
        s = lax.axis_index("s")
        wid = c * 16 + s
        pltpu.sync_copy(z_hbm.at[pl.ds(s * STRIPE, STRIPE)],
                        acc.at[pl.ds(s * STRIPE, STRIPE)])
        pltpu.sync_copy(dst_hbm.at[wid], dst_v)
        pltpu.sync_copy(ones_hbm, ones_v)
        plsc.subcore_barrier()

        def body(j, carry):
            pltpu.sync_copy(ones_v, acc.at[dst_v.at[j]], add=True)
            return carry

        lax.fori_loop(0, K, body, 0)
        plsc.subcore_barrier()
        pltpu.sync_copy(acc.at[pl.ds(s * STRIPE, STRIPE)],
                        out_hbm.at[c, pl.ds(s * STRIPE, STRIPE)])

    return k(dst3, ones_c, z8)


def _wspec(shape):
    return pl.BlockSpec(shape, lambda i: tuple(0 for _ in shape))


def _tc_embed(x_pad, We1, be1, We2, be2):
    def body(x_ref, w1, b1, w2, b2, o_ref):
        h = jnp.tanh(x_ref[...] @ w1[...] + b1[...])
        o_ref[...] = jnp.tanh(h @ w2[...] + b2[...])

    return pl.pallas_call(
        body,
        grid=(NPAD // BT,),
        in_specs=[
            pl.BlockSpec((BT, DIN), lambda i: (i, 0)),
            _wspec((DIN, 64)),
            _wspec((1, 64)),
            _wspec((64, DH)),
            _wspec((1, DH)),
        ],
        out_specs=pl.BlockSpec((BT, DH), lambda i: (i, 0)),
        out_shape=jax.ShapeDtypeStruct((NPAD, DH), jnp.float32),
    )(x_pad, We1, be1, We2, be2)


def _dinv_of(deg_ref):
    d = deg_ref[...]
    dsum = d[0][:, 0:1] + d[1][:, 0:1] + 1.0
    return lax.rsqrt(dsum)


def _tc_pre1(deg, h0, Wg1):
    def body(deg_ref, h_ref, w_ref, o_ref):
        dinv = _dinv_of(deg_ref)
        o_ref[...] = dinv * (h_ref[...] @ w_ref[...])

    return pl.pallas_call(
        body,
        grid=(NPAD // BT,),
        in_specs=[
            pl.BlockSpec((2, BT, DEGW), lambda i: (0, i, 0)),
            pl.BlockSpec((BT, DH), lambda i: (i, 0)),
            _wspec((DH, DH)),
        ],
        out_specs=pl.BlockSpec((BT, DH), lambda i: (i, 0)),
        out_shape=jax.ShapeDtypeStruct((NPAD, DH), jnp.float32),
    )(deg, h0, Wg1)


def _tc_mid(deg, A, g, b, Wnext):
    def body(deg_ref, a_ref, g_ref, b_ref, w_ref, o_ref):
        dinv = _dinv_of(deg_ref)
        a = a_ref[...]
        s = a[0] + a[1] + g_ref[...]
        h = jnp.maximum(dinv * s + b_ref[...], 0.0)
        o_ref[...] = dinv * (h @ w_ref[...])

    return pl.pallas_call(
        body,
        grid=(NPAD // BT,),
        in_specs=[
            pl.BlockSpec((2, BT, DEGW), lambda i: (0, i, 0)),
            pl.BlockSpec((2, BT, DH), lambda i: (0, i, 0)),
            pl.BlockSpec((BT, DH), lambda i: (i, 0)),
            _wspec((1, DH)),
            _wspec((DH, DH)),
        ],
        out_specs=pl.BlockSpec((BT, DH), lambda i: (i, 0)),
        out_shape=jax.ShapeDtypeStruct((NPAD, DH), jnp.float32),
    )(deg, A, g, b, Wnext)


def _tc_final(deg, A, g, bg3, Wp1, bp1, Wp2, bp2, priors_pad):
    def body(deg_ref, a_ref, g_ref, b3_ref, w1_ref, b1_ref, w2_ref, b2_ref,
             p_ref, o_ref):
        dinv = _dinv_of(deg_ref)
        a = a_ref[...]
        s = a[0] + a[1] + g_ref[...]
        h = jnp.maximum(dinv * s + b3_ref[...], 0.0)
        t = jnp.tanh(h @ w1_ref[...] + b1_ref[...])
        o_ref[...] = jnp.tanh(t @ w2_ref[...] + b2_ref[...]) + p_ref[...]

    return pl.pallas_call(
        body,
        grid=(NPAD // BT,),
        in_specs=[
            pl.BlockSpec((2, BT, DEGW), lambda i: (0, i, 0)),
            pl.BlockSpec((2, BT, DH), lambda i: (0, i, 0)),
            pl.BlockSpec((BT, DH), lambda i: (i, 0)),
            _wspec((1, DH)),
            _wspec((DH, DH)),
            _wspec((1, DH)),
            _wspec((DH, 16)),
            _wspec((1, 16)),
            pl.BlockSpec((BT, 16), lambda i: (i, 0)),
        ],
        out_specs=pl.BlockSpec((BT, 16), lambda i: (i, 0)),
        out_shape=jax.ShapeDtypeStruct((NPAD, 16), jnp.float32),
    )(deg, A, g, bg3, Wp1, bp1, Wp2, bp2, priors_pad)


def kernel(x, edge_index, priors, We1, be1, We2, be2, Wg1, bg1, Wg2, bg2,
           Wg3, bg3, Wp1, bp1, Wp2, bp2):
    x_pad = jnp.pad(x, ((0, NPAD - N), (0, 0)))
    priors_pad = jnp.pad(priors, ((0, NPAD - N), (0, 0)))
    src3 = edge_index[0].reshape(NW, K, C)
    dst3 = edge_index[1].reshape(NW, K, C)
    z32 = jnp.zeros((NPAD, DH), jnp.float32)
    z8 = jnp.zeros((NPAD, DEGW), jnp.float32)
    ones_c = jnp.ones((C, DEGW), jnp.float32)

    deg = _sc_deg(dst3, ones_c, z8)
    h0 = _tc_embed(x_pad, We1, be1.reshape(1, -1), We2, be2.reshape(1, -1))

    g1 = _tc_pre1(deg, h0, Wg1)
    A1 = _sc_agg(g1, src3, dst3, z32)
    g2 = _tc_mid(deg, A1, g1, bg1.reshape(1, -1), Wg2)
    A2 = _sc_agg(g2, src3, dst3, z32)
    g3 = _tc_mid(deg, A2, g2, bg2.reshape(1, -1), Wg3)
    A3 = _sc_agg(g3, src3, dst3, z32)
    out = _tc_final(deg, A3, g3, bg3.reshape(1, -1), Wp1, bp1.reshape(1, -1),
                    Wp2, bp2.reshape(1, -1), priors_pad)
    return out[:N]


# speedup vs baseline: 37.4163x; 1.3579x over previous
import functools

import jax
import jax.numpy as jnp
from jax import lax
from jax.experimental import pallas as pl
from jax.experimental.pallas import tpu as pltpu
from jax.experimental.pallas import tpu_sc as plsc

N = 10000
NPAD = 10240
E = 320000
NW = 32
K = 80
C = 125
STRIPE = NPAD // 16
DEGW = 8
BT = 1024
DIN = 128
DH = 32


def _sc_mesh():
    return plsc.VectorSubcoreMesh(core_axis_name="c", subcore_axis_name="s")


def _sc_agg(g_pad, src3, dst3, z32):
    @functools.partial(
        pl.kernel,
        out_type=jax.ShapeDtypeStruct((2, NPAD, DH), jnp.float32),
        mesh=_sc_mesh(),
        compiler_params=pltpu.CompilerParams(use_tc_tiling_on_sc=False),
        scratch_types=[
            pltpu.VMEM((K, C), jnp.int32),
            pltpu.VMEM((K, C), jnp.int32),
            pltpu.VMEM((C, DH), jnp.float32),
            pltpu.VMEM((C, DH), jnp.float32),
            pltpu.VMEM_SHARED((NPAD, DH), jnp.float32),
            pltpu.SemaphoreType.DMA,
            pltpu.SemaphoreType.DMA,
        ],
    )
    def k(g_hbm, src_hbm, dst_hbm, z_hbm, out_hbm, src_v, dst_v, r0, r1, acc,
          gs0, gs1):
        c = lax.axis_index("c")
        s = lax.axis_index("s")
        wid = c * 16 + s
        pltpu.sync_copy(z_hbm.at[pl.ds(s * STRIPE, STRIPE)],
                        acc.at[pl.ds(s * STRIPE, STRIPE)])
        pltpu.sync_copy(src_hbm.at[wid], src_v)
        pltpu.sync_copy(dst_hbm.at[wid], dst_v)
        plsc.subcore_barrier()

        pltpu.async_copy(g_hbm.at[src_v.at[0]], r0, gs0)

        def body(jj, carry):
            j = jj * 2
            pltpu.async_copy(g_hbm.at[src_v.at[j + 1]], r1, gs1)
            pltpu.make_async_copy(g_hbm.at[src_v.at[j]], r0, gs0).wait()
            pltpu.sync_copy(r0, acc.at[dst_v.at[j]], add=True)

            @pl.when(j + 2 < K)
            def _():
                pltpu.async_copy(g_hbm.at[src_v.at[j + 2]], r0, gs0)

            pltpu.make_async_copy(g_hbm.at[src_v.at[j + 1]], r1, gs1).wait()
            pltpu.sync_copy(r1, acc.at[dst_v.at[j + 1]], add=True)
            return carry

        lax.fori_loop(0, K // 2, body, 0)
        plsc.subcore_barrier()
        pltpu.sync_copy(acc.at[pl.ds(s * STRIPE, STRIPE)],
                        out_hbm.at[c, pl.ds(s * STRIPE, STRIPE)])

    return k(g_pad, src3, dst3, z32)


def _sc_deg(dst3, ones_c, z8):
    @functools.partial(
        pl.kernel,
        out_type=jax.ShapeDtypeStruct((2, NPAD, DEGW), jnp.float32),
        mesh=_sc_mesh(),
        compiler_params=pltpu.CompilerParams(use_tc_tiling_on_sc=False),
        scratch_types=[
            pltpu.VMEM((K, C), jnp.int32),
            pltpu.VMEM((C, DEGW), jnp.float32),
            pltpu.VMEM_SHARED((NPAD, DEGW), jnp.float32),
        ],
    )
    def k(dst_hbm, ones_hbm, z_hbm, out_hbm, dst_v, ones_v, acc):
        c = lax.axis_index("c")
        s = lax.axis_index("s")
        wid = c * 16 + s
        pltpu.sync_copy(z_hbm.at[pl.ds(s * STRIPE, STRIPE)],
                        acc.at[pl.ds(s * STRIPE, STRIPE)])
        pltpu.sync_copy(dst_hbm.at[wid], dst_v)
        pltpu.sync_copy(ones_hbm, ones_v)
        plsc.subcore_barrier()

        def body(j, carry):
            pltpu.sync_copy(ones_v, acc.at[dst_v.at[j]], add=True)
            return carry

        lax.fori_loop(0, K, body, 0)
        plsc.subcore_barrier()
        pltpu.sync_copy(acc.at[pl.ds(s * STRIPE, STRIPE)],
                        out_hbm.at[c, pl.ds(s * STRIPE, STRIPE)])

    return k(dst3, ones_c, z8)


def _wspec(shape):
    return pl.BlockSpec(shape, lambda i: tuple(0 for _ in shape))


def _tc_embed(x_pad, We1, be1, We2, be2):
    def body(x_ref, w1, b1, w2, b2, o_ref):
        h = jnp.tanh(x_ref[...] @ w1[...] + b1[...])
        o_ref[...] = jnp.tanh(h @ w2[...] + b2[...])

    return pl.pallas_call(
        body,
        grid=(NPAD // BT,),
        in_specs=[
            pl.BlockSpec((BT, DIN), lambda i: (i, 0)),
            _wspec((DIN, 64)),
            _wspec((1, 64)),
            _wspec((64, DH)),
            _wspec((1, DH)),
        ],
        out_specs=pl.BlockSpec((BT, DH), lambda i: (i, 0)),
        out_shape=jax.ShapeDtypeStruct((NPAD, DH), jnp.float32),
    )(x_pad, We1, be1, We2, be2)


def _dinv_of(deg_ref):
    d = deg_ref[...]
    dsum = d[0][:, 0:1] + d[1][:, 0:1] + 1.0
    return lax.rsqrt(dsum)


def _tc_pre1(deg, h0, Wg1):
    def body(deg_ref, h_ref, w_ref, o_ref):
        dinv = _dinv_of(deg_ref)
        o_ref[...] = dinv * (h_ref[...] @ w_ref[...])

    return pl.pallas_call(
        body,
        grid=(NPAD // BT,),
        in_specs=[
            pl.BlockSpec((2, BT, DEGW), lambda i: (0, i, 0)),
            pl.BlockSpec((BT, DH), lambda i: (i, 0)),
            _wspec((DH, DH)),
        ],
        out_specs=pl.BlockSpec((BT, DH), lambda i: (i, 0)),
        out_shape=jax.ShapeDtypeStruct((NPAD, DH), jnp.float32),
    )(deg, h0, Wg1)


def _tc_mid(deg, A, g, b, Wnext):
    def body(deg_ref, a_ref, g_ref, b_ref, w_ref, o_ref):
        dinv = _dinv_of(deg_ref)
        a = a_ref[...]
        s = a[0] + a[1] + g_ref[...]
        h = jnp.maximum(dinv * s + b_ref[...], 0.0)
        o_ref[...] = dinv * (h @ w_ref[...])

    return pl.pallas_call(
        body,
        grid=(NPAD // BT,),
        in_specs=[
            pl.BlockSpec((2, BT, DEGW), lambda i: (0, i, 0)),
            pl.BlockSpec((2, BT, DH), lambda i: (0, i, 0)),
            pl.BlockSpec((BT, DH), lambda i: (i, 0)),
            _wspec((1, DH)),
            _wspec((DH, DH)),
        ],
        out_specs=pl.BlockSpec((BT, DH), lambda i: (i, 0)),
        out_shape=jax.ShapeDtypeStruct((NPAD, DH), jnp.float32),
    )(deg, A, g, b, Wnext)


def _tc_final(deg, A, g, bg3, Wp1, bp1, Wp2, bp2, priors_pad):
    def body(deg_ref, a_ref, g_ref, b3_ref, w1_ref, b1_ref, w2_ref, b2_ref,
             p_ref, o_ref):
        dinv = _dinv_of(deg_ref)
        a = a_ref[...]
        s = a[0] + a[1] + g_ref[...]
        h = jnp.maximum(dinv * s + b3_ref[...], 0.0)
        t = jnp.tanh(h @ w1_ref[...] + b1_ref[...])
        o_ref[...] = jnp.tanh(t @ w2_ref[...] + b2_ref[...]) + p_ref[...]

    return pl.pallas_call(
        body,
        grid=(NPAD // BT,),
        in_specs=[
            pl.BlockSpec((2, BT, DEGW), lambda i: (0, i, 0)),
            pl.BlockSpec((2, BT, DH), lambda i: (0, i, 0)),
            pl.BlockSpec((BT, DH), lambda i: (i, 0)),
            _wspec((1, DH)),
            _wspec((DH, DH)),
            _wspec((1, DH)),
            _wspec((DH, 16)),
            _wspec((1, 16)),
            pl.BlockSpec((BT, 16), lambda i: (i, 0)),
        ],
        out_specs=pl.BlockSpec((BT, 16), lambda i: (i, 0)),
        out_shape=jax.ShapeDtypeStruct((NPAD, 16), jnp.float32),
    )(deg, A, g, bg3, Wp1, bp1, Wp2, bp2, priors_pad)


def kernel(x, edge_index, priors, We1, be1, We2, be2, Wg1, bg1, Wg2, bg2,
           Wg3, bg3, Wp1, bp1, Wp2, bp2):
    x_pad = jnp.pad(x, ((0, NPAD - N), (0, 0)))
    priors_pad = jnp.pad(priors, ((0, NPAD - N), (0, 0)))
    src3 = edge_index[0].reshape(NW, K, C)
    dst3 = edge_index[1].reshape(NW, K, C)
    z32 = jnp.zeros((NPAD, DH), jnp.float32)
    z8 = jnp.zeros((NPAD, DEGW), jnp.float32)
    ones_c = jnp.ones((C, DEGW), jnp.float32)

    deg = _sc_deg(dst3, ones_c, z8)
    h0 = _tc_embed(x_pad, We1, be1.reshape(1, -1), We2, be2.reshape(1, -1))

    g1 = _tc_pre1(deg, h0, Wg1)
    A1 = _sc_agg(g1, src3, dst3, z32)
    g2 = _tc_mid(deg, A1, g1, bg1.reshape(1, -1), Wg2)
    A2 = _sc_agg(g2, src3, dst3, z32)
    g3 = _tc_mid(deg, A2, g2, bg2.reshape(1, -1), Wg3)
    A3 = _sc_agg(g3, src3, dst3, z32)
    out = _tc_final(deg, A3, g3, bg3.reshape(1, -1), Wp1, bp1.reshape(1, -1),
                    Wp2, bp2.reshape(1, -1), priors_pad)
    return out[:N]


# speedup vs baseline: 41.0890x; 1.0982x over previous
import functools

import jax
import jax.numpy as jnp
from jax import lax
from jax.experimental import pallas as pl
from jax.experimental.pallas import tpu as pltpu
from jax.experimental.pallas import tpu_sc as plsc

N = 10000
NPAD = 10240
E = 320000
NW = 32
K = 80
C = 125
STRIPE = NPAD // 16
DEGW = 8
BT = 1024
DIN = 128
DH = 32


def _sc_mesh():
    return plsc.VectorSubcoreMesh(core_axis_name="c", subcore_axis_name="s")


def _sc_agg(g_pad, src3, dst3, z32):
    @functools.partial(
        pl.kernel,
        out_type=jax.ShapeDtypeStruct((2, NPAD, DH), jnp.float32),
        mesh=_sc_mesh(),
        compiler_params=pltpu.CompilerParams(use_tc_tiling_on_sc=False),
        scratch_types=[
            pltpu.VMEM((K, C), jnp.int32),
            pltpu.VMEM((K, C), jnp.int32),
            pltpu.VMEM((C, DH), jnp.float32),
            pltpu.VMEM((C, DH), jnp.float32),
            pltpu.VMEM_SHARED((NPAD, DH), jnp.float32),
            pltpu.VMEM_SHARED((NPAD, DH), jnp.float32),
            pltpu.SemaphoreType.DMA,
            pltpu.SemaphoreType.DMA,
        ],
    )
    def k(g_hbm, src_hbm, dst_hbm, z_hbm, out_hbm, src_v, dst_v, r0, r1, acc,
          g_sp, gs0, gs1):
        c = lax.axis_index("c")
        s = lax.axis_index("s")
        wid = c * 16 + s
        pltpu.sync_copy(z_hbm.at[pl.ds(s * STRIPE, STRIPE)],
                        acc.at[pl.ds(s * STRIPE, STRIPE)])
        pltpu.sync_copy(g_hbm.at[pl.ds(s * STRIPE, STRIPE)],
                        g_sp.at[pl.ds(s * STRIPE, STRIPE)])
        pltpu.sync_copy(src_hbm.at[wid], src_v)
        pltpu.sync_copy(dst_hbm.at[wid], dst_v)
        plsc.subcore_barrier()

        pltpu.async_copy(g_sp.at[src_v.at[0]], r0, gs0)

        def body(jj, carry):
            j = jj * 2
            pltpu.async_copy(g_sp.at[src_v.at[j + 1]], r1, gs1)
            pltpu.make_async_copy(g_sp.at[src_v.at[j]], r0, gs0).wait()
            pltpu.sync_copy(r0, acc.at[dst_v.at[j]], add=True)

            @pl.when(j + 2 < K)
            def _():
                pltpu.async_copy(g_sp.at[src_v.at[j + 2]], r0, gs0)

            pltpu.make_async_copy(g_sp.at[src_v.at[j + 1]], r1, gs1).wait()
            pltpu.sync_copy(r1, acc.at[dst_v.at[j + 1]], add=True)
            return carry

        lax.fori_loop(0, K // 2, body, 0)
        plsc.subcore_barrier()
        pltpu.sync_copy(acc.at[pl.ds(s * STRIPE, STRIPE)],
                        out_hbm.at[c, pl.ds(s * STRIPE, STRIPE)])

    return k(g_pad, src3, dst3, z32)


def _sc_deg(dst3, ones_c, z8):
    @functools.partial(
        pl.kernel,
        out_type=jax.ShapeDtypeStruct((2, NPAD, DEGW), jnp.float32),
        mesh=_sc_mesh(),
        compiler_params=pltpu.CompilerParams(use_tc_tiling_on_sc=False),
        scratch_types=[
            pltpu.VMEM((K, C), jnp.int32),
            pltpu.VMEM((C, DEGW), jnp.float32),
            pltpu.VMEM_SHARED((NPAD, DEGW), jnp.float32),
        ],
    )
    def k(dst_hbm, ones_hbm, z_hbm, out_hbm, dst_v, ones_v, acc):
        c = lax.axis_index("c")
        s = lax.axis_index("s")
        wid = c * 16 + s
        pltpu.sync_copy(z_hbm.at[pl.ds(s * STRIPE, STRIPE)],
                        acc.at[pl.ds(s * STRIPE, STRIPE)])
        pltpu.sync_copy(dst_hbm.at[wid], dst_v)
        pltpu.sync_copy(ones_hbm, ones_v)
        plsc.subcore_barrier()

        def body(j, carry):
            pltpu.sync_copy(ones_v, acc.at[dst_v.at[j]], add=True)
            return carry

        lax.fori_loop(0, K, body, 0)
        plsc.subcore_barrier()
        pltpu.sync_copy(acc.at[pl.ds(s * STRIPE, STRIPE)],
                        out_hbm.at[c, pl.ds(s * STRIPE, STRIPE)])

    return k(dst3, ones_c, z8)


def _wspec(shape):
    return pl.BlockSpec(shape, lambda i: tuple(0 for _ in shape))


def _tc_embed(x_pad, We1, be1, We2, be2):
    def body(x_ref, w1, b1, w2, b2, o_ref):
        h = jnp.tanh(x_ref[...] @ w1[...] + b1[...])
        o_ref[...] = jnp.tanh(h @ w2[...] + b2[...])

    return pl.pallas_call(
        body,
        grid=(NPAD // BT,),
        in_specs=[
            pl.BlockSpec((BT, DIN), lambda i: (i, 0)),
            _wspec((DIN, 64)),
            _wspec((1, 64)),
            _wspec((64, DH)),
            _wspec((1, DH)),
        ],
        out_specs=pl.BlockSpec((BT, DH), lambda i: (i, 0)),
        out_shape=jax.ShapeDtypeStruct((NPAD, DH), jnp.float32),
    )(x_pad, We1, be1, We2, be2)


def _dinv_of(deg_ref):
    d = deg_ref[...]
    dsum = d[0][:, 0:1] + d[1][:, 0:1] + 1.0
    return lax.rsqrt(dsum)


def _tc_pre1(deg, h0, Wg1):
    def body(deg_ref, h_ref, w_ref, o_ref):
        dinv = _dinv_of(deg_ref)
        o_ref[...] = dinv * (h_ref[...] @ w_ref[...])

    return pl.pallas_call(
        body,
        grid=(NPAD // BT,),
        in_specs=[
            pl.BlockSpec((2, BT, DEGW), lambda i: (0, i, 0)),
            pl.BlockSpec((BT, DH), lambda i: (i, 0)),
            _wspec((DH, DH)),
        ],
        out_specs=pl.BlockSpec((BT, DH), lambda i: (i, 0)),
        out_shape=jax.ShapeDtypeStruct((NPAD, DH), jnp.float32),
    )(deg, h0, Wg1)


def _tc_mid(deg, A, g, b, Wnext):
    def body(deg_ref, a_ref, g_ref, b_ref, w_ref, o_ref):
        dinv = _dinv_of(deg_ref)
        a = a_ref[...]
        s = a[0] + a[1] + g_ref[...]
        h = jnp.maximum(dinv * s + b_ref[...], 0.0)
        o_ref[...] = dinv * (h @ w_ref[...])

    return pl.pallas_call(
        body,
        grid=(NPAD // BT,),
        in_specs=[
            pl.BlockSpec((2, BT, DEGW), lambda i: (0, i, 0)),
            pl.BlockSpec((2, BT, DH), lambda i: (0, i, 0)),
            pl.BlockSpec((BT, DH), lambda i: (i, 0)),
            _wspec((1, DH)),
            _wspec((DH, DH)),
        ],
        out_specs=pl.BlockSpec((BT, DH), lambda i: (i, 0)),
        out_shape=jax.ShapeDtypeStruct((NPAD, DH), jnp.float32),
    )(deg, A, g, b, Wnext)


def _tc_final(deg, A, g, bg3, Wp1, bp1, Wp2, bp2, priors_pad):
    def body(deg_ref, a_ref, g_ref, b3_ref, w1_ref, b1_ref, w2_ref, b2_ref,
             p_ref, o_ref):
        dinv = _dinv_of(deg_ref)
        a = a_ref[...]
        s = a[0] + a[1] + g_ref[...]
        h = jnp.maximum(dinv * s + b3_ref[...], 0.0)
        t = jnp.tanh(h @ w1_ref[...] + b1_ref[...])
        o_ref[...] = jnp.tanh(t @ w2_ref[...] + b2_ref[...]) + p_ref[...]

    return pl.pallas_call(
        body,
        grid=(NPAD // BT,),
        in_specs=[
            pl.BlockSpec((2, BT, DEGW), lambda i: (0, i, 0)),
            pl.BlockSpec((2, BT, DH), lambda i: (0, i, 0)),
            pl.BlockSpec((BT, DH), lambda i: (i, 0)),
            _wspec((1, DH)),
            _wspec((DH, DH)),
            _wspec((1, DH)),
            _wspec((DH, 16)),
            _wspec((1, 16)),
            pl.BlockSpec((BT, 16), lambda i: (i, 0)),
        ],
        out_specs=pl.BlockSpec((BT, 16), lambda i: (i, 0)),
        out_shape=jax.ShapeDtypeStruct((NPAD, 16), jnp.float32),
    )(deg, A, g, bg3, Wp1, bp1, Wp2, bp2, priors_pad)


def kernel(x, edge_index, priors, We1, be1, We2, be2, Wg1, bg1, Wg2, bg2,
           Wg3, bg3, Wp1, bp1, Wp2, bp2):
    x_pad = jnp.pad(x, ((0, NPAD - N), (0, 0)))
    priors_pad = jnp.pad(priors, ((0, NPAD - N), (0, 0)))
    src3 = edge_index[0].reshape(NW, K, C)
    dst3 = edge_index[1].reshape(NW, K, C)
    z32 = jnp.zeros((NPAD, DH), jnp.float32)
    z8 = jnp.zeros((NPAD, DEGW), jnp.float32)
    ones_c = jnp.ones((C, DEGW), jnp.float32)

    deg = _sc_deg(dst3, ones_c, z8)
    h0 = _tc_embed(x_pad, We1, be1.reshape(1, -1), We2, be2.reshape(1, -1))

    g1 = _tc_pre1(deg, h0, Wg1)
    A1 = _sc_agg(g1, src3, dst3, z32)
    g2 = _tc_mid(deg, A1, g1, bg1.reshape(1, -1), Wg2)
    A2 = _sc_agg(g2, src3, dst3, z32)
    g3 = _tc_mid(deg, A2, g2, bg2.reshape(1, -1), Wg3)
    A3 = _sc_agg(g3, src3, dst3, z32)
    out = _tc_final(deg, A3, g3, bg3.reshape(1, -1), Wp1, bp1.reshape(1, -1),
                    Wp2, bp2.reshape(1, -1), priors_pad)
    return out[:N]


# speedup vs baseline: 43.3263x; 1.0544x over previous
import functools

import jax
import jax.numpy as jnp
from jax import lax
from jax.experimental import pallas as pl
from jax.experimental.pallas import tpu as pltpu
from jax.experimental.pallas import tpu_sc as plsc

N = 10000
NPAD = 10240
E = 320000
NW = 32
K = 80
C = 125
STRIPE = NPAD // 16
DEGW = 8
BT = 1024
DIN = 128
DH = 32


def _sc_mesh():
    return plsc.VectorSubcoreMesh(core_axis_name="c", subcore_axis_name="s")


def _sc_agg(g_pad, src3, dst3, z32):
    @functools.partial(
        pl.kernel,
        out_type=jax.ShapeDtypeStruct((2, NPAD, DH), jnp.float32),
        mesh=_sc_mesh(),
        compiler_params=pltpu.CompilerParams(use_tc_tiling_on_sc=False),
        scratch_types=[
            pltpu.VMEM((K, C), jnp.int32),
            pltpu.VMEM((K, C), jnp.int32),
            pltpu.VMEM((C, DH), jnp.float32),
            pltpu.VMEM((C, DH), jnp.float32),
            pltpu.VMEM((C, DH), jnp.float32),
            pltpu.VMEM((C, DH), jnp.float32),
            pltpu.VMEM_SHARED((NPAD, DH), jnp.float32),
            pltpu.VMEM_SHARED((NPAD, DH), jnp.float32),
            pltpu.SemaphoreType.DMA,
            pltpu.SemaphoreType.DMA,
            pltpu.SemaphoreType.DMA,
            pltpu.SemaphoreType.DMA,
            pltpu.SemaphoreType.DMA,
            pltpu.SemaphoreType.DMA,
            pltpu.SemaphoreType.DMA,
            pltpu.SemaphoreType.DMA,
        ],
    )
    def k(g_hbm, src_hbm, dst_hbm, z_hbm, out_hbm, src_v, dst_v,
          r0, r1, r2, r3, acc, g_sp,
          gsem0, gsem1, gsem2, gsem3, ssem0, ssem1, ssem2, ssem3):
        r = (r0, r1, r2, r3)
        gsem = (gsem0, gsem1, gsem2, gsem3)
        ssem = (ssem0, ssem1, ssem2, ssem3)
        c = lax.axis_index("c")
        s = lax.axis_index("s")
        wid = c * 16 + s
        pltpu.sync_copy(z_hbm.at[pl.ds(s * STRIPE, STRIPE)],
                        acc.at[pl.ds(s * STRIPE, STRIPE)])
        pltpu.sync_copy(g_hbm.at[pl.ds(s * STRIPE, STRIPE)],
                        g_sp.at[pl.ds(s * STRIPE, STRIPE)])
        pltpu.sync_copy(src_hbm.at[wid], src_v)
        pltpu.sync_copy(dst_hbm.at[wid], dst_v)
        plsc.subcore_barrier()

        def gather(j, b):
            pltpu.async_copy(g_sp.at[src_v.at[j]], r[b], gsem[b])

        def wait_gather(j, b):
            pltpu.make_async_copy(g_sp.at[src_v.at[j]], r[b], gsem[b]).wait()

        def scatter(j, b):
            pltpu.async_copy(r[b], acc.at[dst_v.at[j]], ssem[b], add=True)

        def wait_scatter(j, b):
            pltpu.make_async_copy(r[b], acc.at[dst_v.at[j]], ssem[b]).wait()

        gather(0, 0)
        gather(1, 1)
        wait_gather(0, 0); scatter(0, 0); gather(2, 2)
        wait_gather(1, 1); scatter(1, 1); gather(3, 3)
        wait_gather(2, 2); scatter(2, 2); wait_scatter(0, 0); gather(4, 0)
        wait_gather(3, 3); scatter(3, 3); wait_scatter(1, 1); gather(5, 1)

        def body(jj, carry):
            j = jj * 4
            for b in range(4):
                wait_gather(j + b, b)
                scatter(j + b, b)
                wait_scatter(j + b - 2, (b + 2) % 4)
                gather(j + b + 2, (b + 2) % 4)
            return carry

        lax.fori_loop(1, K // 4 - 1, body, 0)

        j = K - 4
        wait_gather(j, 0); scatter(j, 0); wait_scatter(j - 2, 2); gather(j + 2, 2)
        wait_gather(j + 1, 1); scatter(j + 1, 1); wait_scatter(j - 1, 3); gather(j + 3, 3)
        wait_gather(j + 2, 2); scatter(j + 2, 2)
        wait_gather(j + 3, 3); scatter(j + 3, 3)
        wait_scatter(j, 0)
        wait_scatter(j + 1, 1)
        wait_scatter(j + 2, 2)
        wait_scatter(j + 3, 3)
        plsc.subcore_barrier()
        pltpu.sync_copy(acc.at[pl.ds(s * STRIPE, STRIPE)],
                        out_hbm.at[c, pl.ds(s * STRIPE, STRIPE)])

    return k(g_pad, src3, dst3, z32)


def _sc_deg(dst3, ones_c, z8):
    @functools.partial(
        pl.kernel,
        out_type=jax.ShapeDtypeStruct((2, NPAD, DEGW), jnp.float32),
        mesh=_sc_mesh(),
        compiler_params=pltpu.CompilerParams(use_tc_tiling_on_sc=False),
        scratch_types=[
            pltpu.VMEM((K, C), jnp.int32),
            pltpu.VMEM((C, DEGW), jnp.float32),
            pltpu.VMEM_SHARED((NPAD, DEGW), jnp.float32),
            pltpu.SemaphoreType.DMA,
        ],
    )
    def k(dst_hbm, ones_hbm, z_hbm, out_hbm, dst_v, ones_v, acc, sem):
        c = lax.axis_index("c")
        s = lax.axis_index("s")
        wid = c * 16 + s
        pltpu.sync_copy(z_hbm.at[pl.ds(s * STRIPE, STRIPE)],
                        acc.at[pl.ds(s * STRIPE, STRIPE)])
        pltpu.sync_copy(dst_hbm.at[wid], dst_v)
        pltpu.sync_copy(ones_hbm, ones_v)
        plsc.subcore_barrier()

        def body(j, carry):
            @pl.when(j >= 8)
            def _():
                pltpu.make_async_copy(ones_v, acc.at[dst_v.at[0]], sem).wait()

            pltpu.async_copy(ones_v, acc.at[dst_v.at[j]], sem, add=True)
            return carry

        lax.fori_loop(0, K, body, 0)

        def drain(j, carry):
            pltpu.make_async_copy(ones_v, acc.at[dst_v.at[0]], sem).wait()
            return carry

        lax.fori_loop(0, 8, drain, 0)
        plsc.subcore_barrier()
        pltpu.sync_copy(acc.at[pl.ds(s * STRIPE, STRIPE)],
                        out_hbm.at[c, pl.ds(s * STRIPE, STRIPE)])

    return k(dst3, ones_c, z8)


def _wspec(shape):
    return pl.BlockSpec(shape, lambda i: tuple(0 for _ in shape))


def _tc_embed(x_pad, We1, be1, We2, be2):
    def body(x_ref, w1, b1, w2, b2, o_ref):
        h = jnp.tanh(x_ref[...] @ w1[...] + b1[...])
        o_ref[...] = jnp.tanh(h @ w2[...] + b2[...])

    return pl.pallas_call(
        body,
        grid=(NPAD // BT,),
        in_specs=[
            pl.BlockSpec((BT, DIN), lambda i: (i, 0)),
            _wspec((DIN, 64)),
            _wspec((1, 64)),
            _wspec((64, DH)),
            _wspec((1, DH)),
        ],
        out_specs=pl.BlockSpec((BT, DH), lambda i: (i, 0)),
        out_shape=jax.ShapeDtypeStruct((NPAD, DH), jnp.float32),
    )(x_pad, We1, be1, We2, be2)


def _dinv_of(deg_ref):
    d = deg_ref[...]
    dsum = d[0][:, 0:1] + d[1][:, 0:1] + 1.0
    return lax.rsqrt(dsum)


def _tc_pre1(deg, h0, Wg1):
    def body(deg_ref, h_ref, w_ref, o_ref):
        dinv = _dinv_of(deg_ref)
        o_ref[...] = dinv * (h_ref[...] @ w_ref[...])

    return pl.pallas_call(
        body,
        grid=(NPAD // BT,),
        in_specs=[
            pl.BlockSpec((2, BT, DEGW), lambda i: (0, i, 0)),
            pl.BlockSpec((BT, DH), lambda i: (i, 0)),
            _wspec((DH, DH)),
        ],
        out_specs=pl.BlockSpec((BT, DH), lambda i: (i, 0)),
        out_shape=jax.ShapeDtypeStruct((NPAD, DH), jnp.float32),
    )(deg, h0, Wg1)


def _tc_mid(deg, A, g, b, Wnext):
    def body(deg_ref, a_ref, g_ref, b_ref, w_ref, o_ref):
        dinv = _dinv_of(deg_ref)
        a = a_ref[...]
        s = a[0] + a[1] + g_ref[...]
        h = jnp.maximum(dinv * s + b_ref[...], 0.0)
        o_ref[...] = dinv * (h @ w_ref[...])

    return pl.pallas_call(
        body,
        grid=(NPAD // BT,),
        in_specs=[
            pl.BlockSpec((2, BT, DEGW), lambda i: (0, i, 0)),
            pl.BlockSpec((2, BT, DH), lambda i: (0, i, 0)),
            pl.BlockSpec((BT, DH), lambda i: (i, 0)),
            _wspec((1, DH)),
            _wspec((DH, DH)),
        ],
        out_specs=pl.BlockSpec((BT, DH), lambda i: (i, 0)),
        out_shape=jax.ShapeDtypeStruct((NPAD, DH), jnp.float32),
    )(deg, A, g, b, Wnext)


def _tc_final(deg, A, g, bg3, Wp1, bp1, Wp2, bp2, priors_pad):
    def body(deg_ref, a_ref, g_ref, b3_ref, w1_ref, b1_ref, w2_ref, b2_ref,
             p_ref, o_ref):
        dinv = _dinv_of(deg_ref)
        a = a_ref[...]
        s = a[0] + a[1] + g_ref[...]
        h = jnp.maximum(dinv * s + b3_ref[...], 0.0)
        t = jnp.tanh(h @ w1_ref[...] + b1_ref[...])
        o_ref[...] = jnp.tanh(t @ w2_ref[...] + b2_ref[...]) + p_ref[...]

    return pl.pallas_call(
        body,
        grid=(NPAD // BT,),
        in_specs=[
            pl.BlockSpec((2, BT, DEGW), lambda i: (0, i, 0)),
            pl.BlockSpec((2, BT, DH), lambda i: (0, i, 0)),
            pl.BlockSpec((BT, DH), lambda i: (i, 0)),
            _wspec((1, DH)),
            _wspec((DH, DH)),
            _wspec((1, DH)),
            _wspec((DH, 16)),
            _wspec((1, 16)),
            pl.BlockSpec((BT, 16), lambda i: (i, 0)),
        ],
        out_specs=pl.BlockSpec((BT, 16), lambda i: (i, 0)),
        out_shape=jax.ShapeDtypeStruct((NPAD, 16), jnp.float32),
    )(deg, A, g, bg3, Wp1, bp1, Wp2, bp2, priors_pad)


def kernel(x, edge_index, priors, We1, be1, We2, be2, Wg1, bg1, Wg2, bg2,
           Wg3, bg3, Wp1, bp1, Wp2, bp2):
    x_pad = jnp.pad(x, ((0, NPAD - N), (0, 0)))
    priors_pad = jnp.pad(priors, ((0, NPAD - N), (0, 0)))
    src3 = edge_index[0].reshape(NW, K, C)
    dst3 = edge_index[1].reshape(NW, K, C)
    z32 = jnp.zeros((NPAD, DH), jnp.float32)
    z8 = jnp.zeros((NPAD, DEGW), jnp.float32)
    ones_c = jnp.ones((C, DEGW), jnp.float32)

    deg = _sc_deg(dst3, ones_c, z8)
    h0 = _tc_embed(x_pad, We1, be1.reshape(1, -1), We2, be2.reshape(1, -1))

    g1 = _tc_pre1(deg, h0, Wg1)
    A1 = _sc_agg(g1, src3, dst3, z32)
    g2 = _tc_mid(deg, A1, g1, bg1.reshape(1, -1), Wg2)
    A2 = _sc_agg(g2, src3, dst3, z32)
    g3 = _tc_mid(deg, A2, g2, bg2.reshape(1, -1), Wg3)
    A3 = _sc_agg(g3, src3, dst3, z32)
    out = _tc_final(deg, A3, g3, bg3.reshape(1, -1), Wp1, bp1.reshape(1, -1),
                    Wp2, bp2.reshape(1, -1), priors_pad)
    return out[:N]


# speedup vs baseline: 46.0821x; 1.0636x over previous
import functools

import jax
import jax.numpy as jnp
from jax import lax
from jax.experimental import pallas as pl
from jax.experimental.pallas import tpu as pltpu
from jax.experimental.pallas import tpu_sc as plsc

N = 10000
NPAD = 10240
E = 320000
NW = 32
K = 80
C = 125
STRIPE = NPAD // 16
DEGW = 8
BT = 2000
DIN = 128
DH = 32


def _sc_mesh():
    return plsc.VectorSubcoreMesh(core_axis_name="c", subcore_axis_name="s")


def _sc_agg(g_pad, src3, dst3, z32):
    @functools.partial(
        pl.kernel,
        out_type=jax.ShapeDtypeStruct((2, NPAD, DH), jnp.float32),
        mesh=_sc_mesh(),
        compiler_params=pltpu.CompilerParams(use_tc_tiling_on_sc=False),
        scratch_types=[
            pltpu.VMEM((K, C), jnp.int32),
            pltpu.VMEM((K, C), jnp.int32),
            pltpu.VMEM((C, DH), jnp.float32),
            pltpu.VMEM((C, DH), jnp.float32),
            pltpu.VMEM((C, DH), jnp.float32),
            pltpu.VMEM((C, DH), jnp.float32),
            pltpu.VMEM_SHARED((NPAD, DH), jnp.float32),
            pltpu.VMEM_SHARED((NPAD, DH), jnp.float32),
            pltpu.SemaphoreType.DMA,
            pltpu.SemaphoreType.DMA,
            pltpu.SemaphoreType.DMA,
            pltpu.SemaphoreType.DMA,
            pltpu.SemaphoreType.DMA,
            pltpu.SemaphoreType.DMA,
            pltpu.SemaphoreType.DMA,
            pltpu.SemaphoreType.DMA,
        ],
    )
    def k(g_hbm, src_hbm, dst_hbm, z_hbm, out_hbm, src_v, dst_v,
          r0, r1, r2, r3, acc, g_sp,
          gsem0, gsem1, gsem2, gsem3, ssem0, ssem1, ssem2, ssem3):
        r = (r0, r1, r2, r3)
        gsem = (gsem0, gsem1, gsem2, gsem3)
        ssem = (ssem0, ssem1, ssem2, ssem3)
        c = lax.axis_index("c")
        s = lax.axis_index("s")
        wid = c * 16 + s
        pltpu.sync_copy(z_hbm.at[pl.ds(s * STRIPE, STRIPE)],
                        acc.at[pl.ds(s * STRIPE, STRIPE)])
        pltpu.sync_copy(g_hbm.at[pl.ds(s * STRIPE, STRIPE)],
                        g_sp.at[pl.ds(s * STRIPE, STRIPE)])
        pltpu.sync_copy(src_hbm.at[wid], src_v)
        pltpu.sync_copy(dst_hbm.at[wid], dst_v)
        plsc.subcore_barrier()

        def gather(j, b):
            pltpu.async_copy(g_sp.at[src_v.at[j]], r[b], gsem[b])

        def wait_gather(j, b):
            pltpu.make_async_copy(g_sp.at[src_v.at[j]], r[b], gsem[b]).wait()

        def scatter(j, b):
            pltpu.async_copy(r[b], acc.at[dst_v.at[j]], ssem[b], add=True)

        def wait_scatter(j, b):
            pltpu.make_async_copy(r[b], acc.at[dst_v.at[j]], ssem[b]).wait()

        gather(0, 0)
        gather(1, 1)
        wait_gather(0, 0); scatter(0, 0); gather(2, 2)
        wait_gather(1, 1); scatter(1, 1); gather(3, 3)
        wait_gather(2, 2); scatter(2, 2); wait_scatter(0, 0); gather(4, 0)
        wait_gather(3, 3); scatter(3, 3); wait_scatter(1, 1); gather(5, 1)

        def body(jj, carry):
            j = jj * 4
            for b in range(4):
                wait_gather(j + b, b)
                scatter(j + b, b)
                wait_scatter(j + b - 2, (b + 2) % 4)
                gather(j + b + 2, (b + 2) % 4)
            return carry

        lax.fori_loop(1, K // 4 - 1, body, 0)

        j = K - 4
        wait_gather(j, 0); scatter(j, 0); wait_scatter(j - 2, 2); gather(j + 2, 2)
        wait_gather(j + 1, 1); scatter(j + 1, 1); wait_scatter(j - 1, 3); gather(j + 3, 3)
        wait_gather(j + 2, 2); scatter(j + 2, 2)
        wait_gather(j + 3, 3); scatter(j + 3, 3)
        wait_scatter(j, 0)
        wait_scatter(j + 1, 1)
        wait_scatter(j + 2, 2)
        wait_scatter(j + 3, 3)
        plsc.subcore_barrier()
        pltpu.sync_copy(acc.at[pl.ds(s * STRIPE, STRIPE)],
                        out_hbm.at[c, pl.ds(s * STRIPE, STRIPE)])

    return k(g_pad, src3, dst3, z32)


def _sc_deg(dst3, ones_c, z8):
    @functools.partial(
        pl.kernel,
        out_type=jax.ShapeDtypeStruct((2, NPAD, DEGW), jnp.float32),
        mesh=_sc_mesh(),
        compiler_params=pltpu.CompilerParams(use_tc_tiling_on_sc=False),
        scratch_types=[
            pltpu.VMEM((K, C), jnp.int32),
            pltpu.VMEM((C, DEGW), jnp.float32),
            pltpu.VMEM_SHARED((NPAD, DEGW), jnp.float32),
            pltpu.SemaphoreType.DMA,
        ],
    )
    def k(dst_hbm, ones_hbm, z_hbm, out_hbm, dst_v, ones_v, acc, sem):
        c = lax.axis_index("c")
        s = lax.axis_index("s")
        wid = c * 16 + s
        pltpu.sync_copy(z_hbm.at[pl.ds(s * STRIPE, STRIPE)],
                        acc.at[pl.ds(s * STRIPE, STRIPE)])
        pltpu.sync_copy(dst_hbm.at[wid], dst_v)
        pltpu.sync_copy(ones_hbm, ones_v)
        plsc.subcore_barrier()

        def body(j, carry):
            @pl.when(j >= 8)
            def _():
                pltpu.make_async_copy(ones_v, acc.at[dst_v.at[0]], sem).wait()

            pltpu.async_copy(ones_v, acc.at[dst_v.at[j]], sem, add=True)
            return carry

        lax.fori_loop(0, K, body, 0)

        def drain(j, carry):
            pltpu.make_async_copy(ones_v, acc.at[dst_v.at[0]], sem).wait()
            return carry

        lax.fori_loop(0, 8, drain, 0)
        plsc.subcore_barrier()
        pltpu.sync_copy(acc.at[pl.ds(s * STRIPE, STRIPE)],
                        out_hbm.at[c, pl.ds(s * STRIPE, STRIPE)])

    return k(dst3, ones_c, z8)


def _wspec(shape):
    return pl.BlockSpec(shape, lambda i: tuple(0 for _ in shape))


def _tc_embed(x, We1, be1, We2, be2):
    def body(x_ref, w1, b1, w2, b2, o_ref):
        h = jnp.tanh(x_ref[...] @ w1[...] + b1[...])
        o_ref[...] = jnp.tanh(h @ w2[...] + b2[...])

    return pl.pallas_call(
        body,
        grid=(N // BT,),
        in_specs=[
            pl.BlockSpec((BT, DIN), lambda i: (i, 0)),
            _wspec((DIN, 64)),
            _wspec((1, 64)),
            _wspec((64, DH)),
            _wspec((1, DH)),
        ],
        out_specs=pl.BlockSpec((BT, DH), lambda i: (i, 0)),
        out_shape=jax.ShapeDtypeStruct((N, DH), jnp.float32),
    )(x, We1, be1, We2, be2)


def _dinv_of(deg_ref):
    d = deg_ref[...]
    dsum = d[0][:, 0:1] + d[1][:, 0:1] + 1.0
    return lax.rsqrt(dsum)


def _tc_pre1(deg, h0, Wg1):
    def body(deg_ref, h_ref, w_ref, o_ref):
        dinv = _dinv_of(deg_ref)
        o_ref[...] = dinv * (h_ref[...] @ w_ref[...])

    return pl.pallas_call(
        body,
        grid=(N // BT,),
        in_specs=[
            pl.BlockSpec((2, BT, DEGW), lambda i: (0, i, 0)),
            pl.BlockSpec((BT, DH), lambda i: (i, 0)),
            _wspec((DH, DH)),
        ],
        out_specs=pl.BlockSpec((BT, DH), lambda i: (i, 0)),
        out_shape=jax.ShapeDtypeStruct((NPAD, DH), jnp.float32),
    )(deg, h0, Wg1)


def _tc_mid(deg, A, g, b, Wnext):
    def body(deg_ref, a_ref, g_ref, b_ref, w_ref, o_ref):
        dinv = _dinv_of(deg_ref)
        a = a_ref[...]
        s = a[0] + a[1] + g_ref[...]
        h = jnp.maximum(dinv * s + b_ref[...], 0.0)
        o_ref[...] = dinv * (h @ w_ref[...])

    return pl.pallas_call(
        body,
        grid=(N // BT,),
        in_specs=[
            pl.BlockSpec((2, BT, DEGW), lambda i: (0, i, 0)),
            pl.BlockSpec((2, BT, DH), lambda i: (0, i, 0)),
            pl.BlockSpec((BT, DH), lambda i: (i, 0)),
            _wspec((1, DH)),
            _wspec((DH, DH)),
        ],
        out_specs=pl.BlockSpec((BT, DH), lambda i: (i, 0)),
        out_shape=jax.ShapeDtypeStruct((NPAD, DH), jnp.float32),
    )(deg, A, g, b, Wnext)


def _tc_final(deg, A, g, bg3, Wp1, bp1, Wp2, bp2, priors):
    def body(deg_ref, a_ref, g_ref, b3_ref, w1_ref, b1_ref, w2_ref, b2_ref,
             p_ref, o_ref):
        dinv = _dinv_of(deg_ref)
        a = a_ref[...]
        s = a[0] + a[1] + g_ref[...]
        h = jnp.maximum(dinv * s + b3_ref[...], 0.0)
        t = jnp.tanh(h @ w1_ref[...] + b1_ref[...])
        o_ref[...] = jnp.tanh(t @ w2_ref[...] + b2_ref[...]) + p_ref[...]

    return pl.pallas_call(
        body,
        grid=(N // BT,),
        in_specs=[
            pl.BlockSpec((2, BT, DEGW), lambda i: (0, i, 0)),
            pl.BlockSpec((2, BT, DH), lambda i: (0, i, 0)),
            pl.BlockSpec((BT, DH), lambda i: (i, 0)),
            _wspec((1, DH)),
            _wspec((DH, DH)),
            _wspec((1, DH)),
            _wspec((DH, 16)),
            _wspec((1, 16)),
            pl.BlockSpec((BT, 16), lambda i: (i, 0)),
        ],
        out_specs=pl.BlockSpec((BT, 16), lambda i: (i, 0)),
        out_shape=jax.ShapeDtypeStruct((N, 16), jnp.float32),
    )(deg, A, g, bg3, Wp1, bp1, Wp2, bp2, priors)


def kernel(x, edge_index, priors, We1, be1, We2, be2, Wg1, bg1, Wg2, bg2,
           Wg3, bg3, Wp1, bp1, Wp2, bp2):
    src3 = edge_index[0].reshape(NW, K, C)
    dst3 = edge_index[1].reshape(NW, K, C)
    z32 = jnp.zeros((NPAD, DH), jnp.float32)
    z8 = jnp.zeros((NPAD, DEGW), jnp.float32)
    ones_c = jnp.ones((C, DEGW), jnp.float32)

    deg = _sc_deg(dst3, ones_c, z8)
    h0 = _tc_embed(x, We1, be1.reshape(1, -1), We2, be2.reshape(1, -1))

    g1 = _tc_pre1(deg, h0, Wg1)
    A1 = _sc_agg(g1, src3, dst3, z32)
    g2 = _tc_mid(deg, A1, g1, bg1.reshape(1, -1), Wg2)
    A2 = _sc_agg(g2, src3, dst3, z32)
    g3 = _tc_mid(deg, A2, g2, bg2.reshape(1, -1), Wg3)
    A3 = _sc_agg(g3, src3, dst3, z32)
    out = _tc_final(deg, A3, g3, bg3.reshape(1, -1), Wp1, bp1.reshape(1, -1),
                    Wp2, bp2.reshape(1, -1), priors)
    return out


# speedup vs baseline: 46.3909x; 1.0067x over previous
import functools

import jax
import jax.numpy as jnp
from jax import lax
from jax.experimental import pallas as pl
from jax.experimental.pallas import tpu as pltpu
from jax.experimental.pallas import tpu_sc as plsc

N = 10000
NPAD = 10240
E = 320000
NW = 32
K = 80
C = 125
CB = 1000
KB = (E // NW) // CB
STRIPE = NPAD // 16
DEGW = 8
BT = 2000
DIN = 128
DH = 32


def _sc_mesh():
    return plsc.VectorSubcoreMesh(core_axis_name="c", subcore_axis_name="s")


def _sc_agg(g_pad, src3, dst3, z32):
    @functools.partial(
        pl.kernel,
        out_type=jax.ShapeDtypeStruct((2, NPAD, DH), jnp.float32),
        mesh=_sc_mesh(),
        compiler_params=pltpu.CompilerParams(use_tc_tiling_on_sc=False),
        scratch_types=[
            pltpu.VMEM((KB, CB), jnp.int32),
            pltpu.VMEM((KB, CB), jnp.int32),
            pltpu.VMEM((CB, DH), jnp.float32),
            pltpu.VMEM((CB, DH), jnp.float32),
            pltpu.VMEM_SHARED((NPAD, DH), jnp.float32),
            pltpu.VMEM_SHARED((NPAD, DH), jnp.float32),
            pltpu.SemaphoreType.DMA,
            pltpu.SemaphoreType.DMA,
            pltpu.SemaphoreType.DMA,
            pltpu.SemaphoreType.DMA,
        ],
    )
    def k(g_hbm, src_hbm, dst_hbm, z_hbm, out_hbm, src_v, dst_v,
          r0, r1, acc, g_sp, gsem0, gsem1, ssem0, ssem1):
        r = (r0, r1)
        gsem = (gsem0, gsem1)
        ssem = (ssem0, ssem1)
        c = lax.axis_index("c")
        s = lax.axis_index("s")
        wid = c * 16 + s
        pltpu.sync_copy(z_hbm.at[pl.ds(s * STRIPE, STRIPE)],
                        acc.at[pl.ds(s * STRIPE, STRIPE)])
        pltpu.sync_copy(g_hbm.at[pl.ds(s * STRIPE, STRIPE)],
                        g_sp.at[pl.ds(s * STRIPE, STRIPE)])
        pltpu.sync_copy(src_hbm.at[wid], src_v)
        pltpu.sync_copy(dst_hbm.at[wid], dst_v)
        plsc.subcore_barrier()

        def gather(jj, b):
            pltpu.async_copy(g_sp.at[src_v.at[jj]], r[b], gsem[b])

        def wait_gather(jj, b):
            pltpu.make_async_copy(g_sp.at[src_v.at[jj]], r[b], gsem[b]).wait()

        def scatter(jj, b):
            pltpu.async_copy(r[b], acc.at[dst_v.at[jj]], ssem[b], add=True)

        def wait_scatter(jj, b):
            pltpu.make_async_copy(r[b], acc.at[dst_v.at[jj]], ssem[b]).wait()

        gather(0, 0)
        wait_gather(0, 0); gather(1, 1); scatter(0, 0)
        wait_gather(1, 1); wait_scatter(0, 0); gather(2, 0); scatter(1, 1)

        def body(ii, carry):
            jj = ii * 2
            wait_gather(jj, 0); wait_scatter(jj - 1, 1)
            gather(jj + 1, 1); scatter(jj, 0)
            wait_gather(jj + 1, 1); wait_scatter(jj, 0)
            gather(jj + 2, 0); scatter(jj + 1, 1)
            return carry

        lax.fori_loop(1, KB // 2 - 1, body, 0)

        jj = KB - 2
        wait_gather(jj, 0); wait_scatter(jj - 1, 1); gather(jj + 1, 1); scatter(jj, 0)
        wait_gather(jj + 1, 1); wait_scatter(jj, 0); scatter(jj + 1, 1)
        wait_scatter(jj + 1, 1)
        plsc.subcore_barrier()
        pltpu.sync_copy(acc.at[pl.ds(s * STRIPE, STRIPE)],
                        out_hbm.at[c, pl.ds(s * STRIPE, STRIPE)])

    return k(g_pad, src3, dst3, z32)


def _sc_deg(dst3, ones_c, z8):
    @functools.partial(
        pl.kernel,
        out_type=jax.ShapeDtypeStruct((2, NPAD, DEGW), jnp.float32),
        mesh=_sc_mesh(),
        compiler_params=pltpu.CompilerParams(use_tc_tiling_on_sc=False),
        scratch_types=[
            pltpu.VMEM((KB, CB), jnp.int32),
            pltpu.VMEM((CB, DEGW), jnp.float32),
            pltpu.VMEM_SHARED((NPAD, DEGW), jnp.float32),
            pltpu.SemaphoreType.DMA,
        ],
    )
    def k(dst_hbm, ones_hbm, z_hbm, out_hbm, dst_v, ones_v, acc, sem):
        c = lax.axis_index("c")
        s = lax.axis_index("s")
        wid = c * 16 + s
        pltpu.sync_copy(z_hbm.at[pl.ds(s * STRIPE, STRIPE)],
                        acc.at[pl.ds(s * STRIPE, STRIPE)])
        pltpu.sync_copy(dst_hbm.at[wid], dst_v)
        pltpu.sync_copy(ones_hbm, ones_v)
        plsc.subcore_barrier()

        def body(jj, carry):
            pltpu.async_copy(ones_v, acc.at[dst_v.at[jj]], sem, add=True)
            return carry

        lax.fori_loop(0, KB, body, 0)

        def drain(jj, carry):
            pltpu.make_async_copy(ones_v, acc.at[dst_v.at[0]], sem).wait()
            return carry

        lax.fori_loop(0, KB, drain, 0)
        plsc.subcore_barrier()
        pltpu.sync_copy(acc.at[pl.ds(s * STRIPE, STRIPE)],
                        out_hbm.at[c, pl.ds(s * STRIPE, STRIPE)])

    return k(dst3, ones_c, z8)


def _wspec(shape):
    return pl.BlockSpec(shape, lambda i: tuple(0 for _ in shape))


def _tc_embed(x, We1, be1, We2, be2):
    def body(x_ref, w1, b1, w2, b2, o_ref):
        h = jnp.tanh(x_ref[...] @ w1[...] + b1[...])
        o_ref[...] = jnp.tanh(h @ w2[...] + b2[...])

    return pl.pallas_call(
        body,
        grid=(N // BT,),
        in_specs=[
            pl.BlockSpec((BT, DIN), lambda i: (i, 0)),
            _wspec((DIN, 64)),
            _wspec((1, 64)),
            _wspec((64, DH)),
            _wspec((1, DH)),
        ],
        out_specs=pl.BlockSpec((BT, DH), lambda i: (i, 0)),
        out_shape=jax.ShapeDtypeStruct((N, DH), jnp.float32),
    )(x, We1, be1, We2, be2)


def _dinv_of(deg_ref):
    d = deg_ref[...]
    dsum = d[0][:, 0:1] + d[1][:, 0:1] + 1.0
    return lax.rsqrt(dsum)


def _tc_pre1(deg, h0, Wg1):
    def body(deg_ref, h_ref, w_ref, o_ref):
        dinv = _dinv_of(deg_ref)
        o_ref[...] = dinv * (h_ref[...] @ w_ref[...])

    return pl.pallas_call(
        body,
        grid=(N // BT,),
        in_specs=[
            pl.BlockSpec((2, BT, DEGW), lambda i: (0, i, 0)),
            pl.BlockSpec((BT, DH), lambda i: (i, 0)),
            _wspec((DH, DH)),
        ],
        out_specs=pl.BlockSpec((BT, DH), lambda i: (i, 0)),
        out_shape=jax.ShapeDtypeStruct((NPAD, DH), jnp.float32),
    )(deg, h0, Wg1)


def _tc_mid(deg, A, g, b, Wnext):
    def body(deg_ref, a_ref, g_ref, b_ref, w_ref, o_ref):
        dinv = _dinv_of(deg_ref)
        a = a_ref[...]
        s = a[0] + a[1] + g_ref[...]
        h = jnp.maximum(dinv * s + b_ref[...], 0.0)
        o_ref[...] = dinv * (h @ w_ref[...])

    return pl.pallas_call(
        body,
        grid=(N // BT,),
        in_specs=[
            pl.BlockSpec((2, BT, DEGW), lambda i: (0, i, 0)),
            pl.BlockSpec((2, BT, DH), lambda i: (0, i, 0)),
            pl.BlockSpec((BT, DH), lambda i: (i, 0)),
            _wspec((1, DH)),
            _wspec((DH, DH)),
        ],
        out_specs=pl.BlockSpec((BT, DH), lambda i: (i, 0)),
        out_shape=jax.ShapeDtypeStruct((NPAD, DH), jnp.float32),
    )(deg, A, g, b, Wnext)


def _tc_final(deg, A, g, bg3, Wp1, bp1, Wp2, bp2, priors):
    def body(deg_ref, a_ref, g_ref, b3_ref, w1_ref, b1_ref, w2_ref, b2_ref,
             p_ref, o_ref):
        dinv = _dinv_of(deg_ref)
        a = a_ref[...]
        s = a[0] + a[1] + g_ref[...]
        h = jnp.maximum(dinv * s + b3_ref[...], 0.0)
        t = jnp.tanh(h @ w1_ref[...] + b1_ref[...])
        o_ref[...] = jnp.tanh(t @ w2_ref[...] + b2_ref[...]) + p_ref[...]

    return pl.pallas_call(
        body,
        grid=(N // BT,),
        in_specs=[
            pl.BlockSpec((2, BT, DEGW), lambda i: (0, i, 0)),
            pl.BlockSpec((2, BT, DH), lambda i: (0, i, 0)),
            pl.BlockSpec((BT, DH), lambda i: (i, 0)),
            _wspec((1, DH)),
            _wspec((DH, DH)),
            _wspec((1, DH)),
            _wspec((DH, 16)),
            _wspec((1, 16)),
            pl.BlockSpec((BT, 16), lambda i: (i, 0)),
        ],
        out_specs=pl.BlockSpec((BT, 16), lambda i: (i, 0)),
        out_shape=jax.ShapeDtypeStruct((N, 16), jnp.float32),
    )(deg, A, g, bg3, Wp1, bp1, Wp2, bp2, priors)


def kernel(x, edge_index, priors, We1, be1, We2, be2, Wg1, bg1, Wg2, bg2,
           Wg3, bg3, Wp1, bp1, Wp2, bp2):
    src3 = edge_index[0].reshape(NW, KB, CB)
    dst3 = edge_index[1].reshape(NW, KB, CB)
    z32 = jnp.zeros((NPAD, DH), jnp.float32)
    z8 = jnp.zeros((NPAD, DEGW), jnp.float32)
    ones_c = jnp.ones((CB, DEGW), jnp.float32)

    deg = _sc_deg(dst3, ones_c, z8)
    h0 = _tc_embed(x, We1, be1.reshape(1, -1), We2, be2.reshape(1, -1))

    g1 = _tc_pre1(deg, h0, Wg1)
    A1 = _sc_agg(g1, src3, dst3, z32)
    g2 = _tc_mid(deg, A1, g1, bg1.reshape(1, -1), Wg2)
    A2 = _sc_agg(g2, src3, dst3, z32)
    g3 = _tc_mid(deg, A2, g2, bg2.reshape(1, -1), Wg3)
    A3 = _sc_agg(g3, src3, dst3, z32)
    out = _tc_final(deg, A3, g3, bg3.reshape(1, -1), Wp1, bp1.reshape(1, -1),
                    Wp2, bp2.reshape(1, -1), priors)
    return out


# speedup vs baseline: 46.4865x; 1.0021x over previous
import functools

import jax
import jax.numpy as jnp
from jax import lax
from jax.experimental import pallas as pl
from jax.experimental.pallas import tpu as pltpu
from jax.experimental.pallas import tpu_sc as plsc

N = 10000
NPAD = 10240
E = 320000
NW = 32
K = 80
C = 125
CB = 1000
KB = (E // NW) // CB
STRIPE = NPAD // 16
DEGW = 8
BT = 5000
DIN = 128
DH = 32


def _sc_mesh():
    return plsc.VectorSubcoreMesh(core_axis_name="c", subcore_axis_name="s")


def _sc_agg(g_pad, src3, dst3, z32):
    @functools.partial(
        pl.kernel,
        out_type=jax.ShapeDtypeStruct((2, NPAD, DH), jnp.float32),
        mesh=_sc_mesh(),
        compiler_params=pltpu.CompilerParams(use_tc_tiling_on_sc=False),
        scratch_types=[
            pltpu.VMEM((KB, CB), jnp.int32),
            pltpu.VMEM((KB, CB), jnp.int32),
            pltpu.VMEM((CB, DH), jnp.float32),
            pltpu.VMEM((CB, DH), jnp.float32),
            pltpu.VMEM_SHARED((NPAD, DH), jnp.float32),
            pltpu.VMEM_SHARED((NPAD, DH), jnp.float32),
            pltpu.SemaphoreType.DMA,
            pltpu.SemaphoreType.DMA,
            pltpu.SemaphoreType.DMA,
            pltpu.SemaphoreType.DMA,
        ],
    )
    def k(g_hbm, src_hbm, dst_hbm, z_hbm, out_hbm, src_v, dst_v,
          r0, r1, acc, g_sp, gsem0, gsem1, ssem0, ssem1):
        r = (r0, r1)
        gsem = (gsem0, gsem1)
        ssem = (ssem0, ssem1)
        c = lax.axis_index("c")
        s = lax.axis_index("s")
        wid = c * 16 + s
        pltpu.sync_copy(z_hbm.at[pl.ds(s * STRIPE, STRIPE)],
                        acc.at[pl.ds(s * STRIPE, STRIPE)])
        pltpu.sync_copy(g_hbm.at[pl.ds(s * STRIPE, STRIPE)],
                        g_sp.at[pl.ds(s * STRIPE, STRIPE)])
        pltpu.sync_copy(src_hbm.at[wid], src_v)
        pltpu.sync_copy(dst_hbm.at[wid], dst_v)
        plsc.subcore_barrier()

        def gather(jj, b):
            pltpu.async_copy(g_sp.at[src_v.at[jj]], r[b], gsem[b])

        def wait_gather(jj, b):
            pltpu.make_async_copy(g_sp.at[src_v.at[jj]], r[b], gsem[b]).wait()

        def scatter(jj, b):
            pltpu.async_copy(r[b], acc.at[dst_v.at[jj]], ssem[b], add=True)

        def wait_scatter(jj, b):
            pltpu.make_async_copy(r[b], acc.at[dst_v.at[jj]], ssem[b]).wait()

        gather(0, 0)
        wait_gather(0, 0); gather(1, 1); scatter(0, 0)
        wait_gather(1, 1); wait_scatter(0, 0); gather(2, 0); scatter(1, 1)

        def body(ii, carry):
            jj = ii * 2
            wait_gather(jj, 0); wait_scatter(jj - 1, 1)
            gather(jj + 1, 1); scatter(jj, 0)
            wait_gather(jj + 1, 1); wait_scatter(jj, 0)
            gather(jj + 2, 0); scatter(jj + 1, 1)
            return carry

        lax.fori_loop(1, KB // 2 - 1, body, 0)

        jj = KB - 2
        wait_gather(jj, 0); wait_scatter(jj - 1, 1); gather(jj + 1, 1); scatter(jj, 0)
        wait_gather(jj + 1, 1); wait_scatter(jj, 0); scatter(jj + 1, 1)
        wait_scatter(jj + 1, 1)
        plsc.subcore_barrier()
        pltpu.sync_copy(acc.at[pl.ds(s * STRIPE, STRIPE)],
                        out_hbm.at[c, pl.ds(s * STRIPE, STRIPE)])

    return k(g_pad, src3, dst3, z32)


def _sc_deg(dst3, ones_c, z8):
    @functools.partial(
        pl.kernel,
        out_type=jax.ShapeDtypeStruct((2, NPAD, DEGW), jnp.float32),
        mesh=_sc_mesh(),
        compiler_params=pltpu.CompilerParams(use_tc_tiling_on_sc=False),
        scratch_types=[
            pltpu.VMEM((KB, CB), jnp.int32),
            pltpu.VMEM((CB, DEGW), jnp.float32),
            pltpu.VMEM_SHARED((NPAD, DEGW), jnp.float32),
            pltpu.SemaphoreType.DMA,
        ],
    )
    def k(dst_hbm, ones_hbm, z_hbm, out_hbm, dst_v, ones_v, acc, sem):
        c = lax.axis_index("c")
        s = lax.axis_index("s")
        wid = c * 16 + s
        pltpu.sync_copy(z_hbm.at[pl.ds(s * STRIPE, STRIPE)],
                        acc.at[pl.ds(s * STRIPE, STRIPE)])
        pltpu.sync_copy(dst_hbm.at[wid], dst_v)
        pltpu.sync_copy(ones_hbm, ones_v)
        plsc.subcore_barrier()

        def body(jj, carry):
            pltpu.async_copy(ones_v, acc.at[dst_v.at[jj]], sem, add=True)
            return carry

        lax.fori_loop(0, KB, body, 0)

        def drain(jj, carry):
            pltpu.make_async_copy(ones_v, acc.at[dst_v.at[0]], sem).wait()
            return carry

        lax.fori_loop(0, KB, drain, 0)
        plsc.subcore_barrier()
        pltpu.sync_copy(acc.at[pl.ds(s * STRIPE, STRIPE)],
                        out_hbm.at[c, pl.ds(s * STRIPE, STRIPE)])

    return k(dst3, ones_c, z8)


def _wspec(shape):
    return pl.BlockSpec(shape, lambda i: tuple(0 for _ in shape))


def _tc_embed(x, We1, be1, We2, be2):
    def body(x_ref, w1, b1, w2, b2, o_ref):
        h = jnp.tanh(x_ref[...] @ w1[...] + b1[...])
        o_ref[...] = jnp.tanh(h @ w2[...] + b2[...])

    return pl.pallas_call(
        body,
        grid=(N // BT,),
        in_specs=[
            pl.BlockSpec((BT, DIN), lambda i: (i, 0)),
            _wspec((DIN, 64)),
            _wspec((1, 64)),
            _wspec((64, DH)),
            _wspec((1, DH)),
        ],
        out_specs=pl.BlockSpec((BT, DH), lambda i: (i, 0)),
        out_shape=jax.ShapeDtypeStruct((N, DH), jnp.float32),
    )(x, We1, be1, We2, be2)


def _dinv_of(deg_ref):
    d = deg_ref[...]
    dsum = d[0][:, 0:1] + d[1][:, 0:1] + 1.0
    return lax.rsqrt(dsum)


def _tc_pre1(deg, h0, Wg1):
    def body(deg_ref, h_ref, w_ref, o_ref):
        dinv = _dinv_of(deg_ref)
        o_ref[...] = dinv * (h_ref[...] @ w_ref[...])

    return pl.pallas_call(
        body,
        grid=(N // BT,),
        in_specs=[
            pl.BlockSpec((2, BT, DEGW), lambda i: (0, i, 0)),
            pl.BlockSpec((BT, DH), lambda i: (i, 0)),
            _wspec((DH, DH)),
        ],
        out_specs=pl.BlockSpec((BT, DH), lambda i: (i, 0)),
        out_shape=jax.ShapeDtypeStruct((NPAD, DH), jnp.float32),
    )(deg, h0, Wg1)


def _tc_mid(deg, A, g, b, Wnext):
    def body(deg_ref, a_ref, g_ref, b_ref, w_ref, o_ref):
        dinv = _dinv_of(deg_ref)
        a = a_ref[...]
        s = a[0] + a[1] + g_ref[...]
        h = jnp.maximum(dinv * s + b_ref[...], 0.0)
        o_ref[...] = dinv * (h @ w_ref[...])

    return pl.pallas_call(
        body,
        grid=(N // BT,),
        in_specs=[
            pl.BlockSpec((2, BT, DEGW), lambda i: (0, i, 0)),
            pl.BlockSpec((2, BT, DH), lambda i: (0, i, 0)),
            pl.BlockSpec((BT, DH), lambda i: (i, 0)),
            _wspec((1, DH)),
            _wspec((DH, DH)),
        ],
        out_specs=pl.BlockSpec((BT, DH), lambda i: (i, 0)),
        out_shape=jax.ShapeDtypeStruct((NPAD, DH), jnp.float32),
    )(deg, A, g, b, Wnext)


def _tc_final(deg, A, g, bg3, Wp1, bp1, Wp2, bp2, priors):
    def body(deg_ref, a_ref, g_ref, b3_ref, w1_ref, b1_ref, w2_ref, b2_ref,
             p_ref, o_ref):
        dinv = _dinv_of(deg_ref)
        a = a_ref[...]
        s = a[0] + a[1] + g_ref[...]
        h = jnp.maximum(dinv * s + b3_ref[...], 0.0)
        t = jnp.tanh(h @ w1_ref[...] + b1_ref[...])
        o_ref[...] = jnp.tanh(t @ w2_ref[...] + b2_ref[...]) + p_ref[...]

    return pl.pallas_call(
        body,
        grid=(N // BT,),
        in_specs=[
            pl.BlockSpec((2, BT, DEGW), lambda i: (0, i, 0)),
            pl.BlockSpec((2, BT, DH), lambda i: (0, i, 0)),
            pl.BlockSpec((BT, DH), lambda i: (i, 0)),
            _wspec((1, DH)),
            _wspec((DH, DH)),
            _wspec((1, DH)),
            _wspec((DH, 16)),
            _wspec((1, 16)),
            pl.BlockSpec((BT, 16), lambda i: (i, 0)),
        ],
        out_specs=pl.BlockSpec((BT, 16), lambda i: (i, 0)),
        out_shape=jax.ShapeDtypeStruct((N, 16), jnp.float32),
    )(deg, A, g, bg3, Wp1, bp1, Wp2, bp2, priors)


def kernel(x, edge_index, priors, We1, be1, We2, be2, Wg1, bg1, Wg2, bg2,
           Wg3, bg3, Wp1, bp1, Wp2, bp2):
    src3 = edge_index[0].reshape(NW, KB, CB)
    dst3 = edge_index[1].reshape(NW, KB, CB)
    z32 = jnp.zeros((NPAD, DH), jnp.float32)
    z8 = jnp.zeros((NPAD, DEGW), jnp.float32)
    ones_c = jnp.ones((CB, DEGW), jnp.float32)

    deg = _sc_deg(dst3, ones_c, z8)
    h0 = _tc_embed(x, We1, be1.reshape(1, -1), We2, be2.reshape(1, -1))

    g1 = _tc_pre1(deg, h0, Wg1)
    A1 = _sc_agg(g1, src3, dst3, z32)
    g2 = _tc_mid(deg, A1, g1, bg1.reshape(1, -1), Wg2)
    A2 = _sc_agg(g2, src3, dst3, z32)
    g3 = _tc_mid(deg, A2, g2, bg2.reshape(1, -1), Wg3)
    A3 = _sc_agg(g3, src3, dst3, z32)
    out = _tc_final(deg, A3, g3, bg3.reshape(1, -1), Wp1, bp1.reshape(1, -1),
                    Wp2, bp2.reshape(1, -1), priors)
    return out


# speedup vs baseline: 46.5393x; 1.0011x over previous
import functools

import jax
import jax.numpy as jnp
from jax import lax
from jax.experimental import pallas as pl
from jax.experimental.pallas import tpu as pltpu
from jax.experimental.pallas import tpu_sc as plsc

N = 10000
NPAD = 10240
E = 320000
NW = 32
CB = 1000
KB = (E // NW) // CB
STRIPE = NPAD // 16
DEGW = 8
BT = 5000
DIN = 128
DH = 32


def _sc_mesh():
    return plsc.VectorSubcoreMesh(core_axis_name="c", subcore_axis_name="s")


def _sc_agg(g_pad, src3, dst3, z32):
    @functools.partial(
        pl.kernel,
        out_type=jax.ShapeDtypeStruct((2, NPAD, DH), jnp.float32),
        mesh=_sc_mesh(),
        compiler_params=pltpu.CompilerParams(use_tc_tiling_on_sc=False),
        scratch_types=[
            pltpu.VMEM((KB, CB), jnp.int32),
            pltpu.VMEM((KB, CB), jnp.int32),
            pltpu.VMEM((CB, DH), jnp.float32),
            pltpu.VMEM((CB, DH), jnp.float32),
            pltpu.VMEM_SHARED((NPAD, DH), jnp.float32),
            pltpu.VMEM_SHARED((NPAD, DH), jnp.float32),
            pltpu.SemaphoreType.DMA,
            pltpu.SemaphoreType.DMA,
            pltpu.SemaphoreType.DMA,
            pltpu.SemaphoreType.DMA,
        ],
    )
    def k(g_hbm, src_hbm, dst_hbm, z_hbm, out_hbm, src_v, dst_v,
          r0, r1, acc, g_sp, gsem0, gsem1, ssem0, ssem1):
        r = (r0, r1)
        gsem = (gsem0, gsem1)
        ssem = (ssem0, ssem1)
        c = lax.axis_index("c")
        s = lax.axis_index("s")
        wid = c * 16 + s
        pltpu.sync_copy(z_hbm.at[pl.ds(s * STRIPE, STRIPE)],
                        acc.at[pl.ds(s * STRIPE, STRIPE)])
        pltpu.sync_copy(g_hbm.at[pl.ds(s * STRIPE, STRIPE)],
                        g_sp.at[pl.ds(s * STRIPE, STRIPE)])
        pltpu.sync_copy(src_hbm.at[wid], src_v)
        pltpu.sync_copy(dst_hbm.at[wid], dst_v)
        plsc.subcore_barrier()

        def gather(jj, b):
            pltpu.async_copy(g_sp.at[src_v.at[jj]], r[b], gsem[b])

        def wait_gather(jj, b):
            pltpu.make_async_copy(g_sp.at[src_v.at[jj]], r[b], gsem[b]).wait()

        def scatter(jj, b):
            pltpu.async_copy(r[b], acc.at[dst_v.at[jj]], ssem[b], add=True)

        def wait_scatter(jj, b):
            pltpu.make_async_copy(r[b], acc.at[dst_v.at[jj]], ssem[b]).wait()

        gather(0, 0)
        wait_gather(0, 0); gather(1, 1); scatter(0, 0)
        wait_gather(1, 1); wait_scatter(0, 0); gather(2, 0); scatter(1, 1)

        def body(ii, carry):
            jj = ii * 2
            wait_gather(jj, 0); wait_scatter(jj - 1, 1)
            gather(jj + 1, 1); scatter(jj, 0)
            wait_gather(jj + 1, 1); wait_scatter(jj, 0)
            gather(jj + 2, 0); scatter(jj + 1, 1)
            return carry

        lax.fori_loop(1, KB // 2 - 1, body, 0)

        jj = KB - 2
        wait_gather(jj, 0); wait_scatter(jj - 1, 1); gather(jj + 1, 1); scatter(jj, 0)
        wait_gather(jj + 1, 1); wait_scatter(jj, 0); scatter(jj + 1, 1)
        wait_scatter(jj + 1, 1)
        plsc.subcore_barrier()
        pltpu.sync_copy(acc.at[pl.ds(s * STRIPE, STRIPE)],
                        out_hbm.at[c, pl.ds(s * STRIPE, STRIPE)])

    return k(g_pad, src3, dst3, z32)


def _sc_deg(dst3, ones_c, z8):
    @functools.partial(
        pl.kernel,
        out_type=jax.ShapeDtypeStruct((2, NPAD, DEGW), jnp.float32),
        mesh=_sc_mesh(),
        compiler_params=pltpu.CompilerParams(use_tc_tiling_on_sc=False),
        scratch_types=[
            pltpu.VMEM((KB, CB), jnp.int32),
            pltpu.VMEM((CB, DEGW), jnp.float32),
            pltpu.VMEM_SHARED((NPAD, DEGW), jnp.float32),
            pltpu.SemaphoreType.DMA,
        ],
    )
    def k(dst_hbm, ones_hbm, z_hbm, out_hbm, dst_v, ones_v, acc, sem):
        c = lax.axis_index("c")
        s = lax.axis_index("s")
        wid = c * 16 + s
        pltpu.sync_copy(z_hbm.at[pl.ds(s * STRIPE, STRIPE)],
                        acc.at[pl.ds(s * STRIPE, STRIPE)])
        pltpu.sync_copy(dst_hbm.at[wid], dst_v)
        pltpu.sync_copy(ones_hbm, ones_v)
        plsc.subcore_barrier()

        def body(jj, carry):
            pltpu.async_copy(ones_v, acc.at[dst_v.at[jj]], sem, add=True)
            return carry

        lax.fori_loop(0, KB, body, 0)

        def drain(jj, carry):
            pltpu.make_async_copy(ones_v, acc.at[dst_v.at[0]], sem).wait()
            return carry

        lax.fori_loop(0, KB, drain, 0)
        plsc.subcore_barrier()
        pltpu.sync_copy(acc.at[pl.ds(s * STRIPE, STRIPE)],
                        out_hbm.at[c, pl.ds(s * STRIPE, STRIPE)])

    return k(dst3, ones_c, z8)


def _wspec(shape):
    return pl.BlockSpec(shape, lambda i: tuple(0 for _ in shape))


def _tc_embed(x, We1, be1, We2, be2):
    def body(x_ref, w1, b1, w2, b2, o_ref):
        h = jnp.tanh(x_ref[...] @ w1[...] + b1[...])
        o_ref[...] = jnp.tanh(h @ w2[...] + b2[...])

    return pl.pallas_call(
        body,
        grid=(N // BT,),
        in_specs=[
            pl.BlockSpec((BT, DIN), lambda i: (i, 0)),
            _wspec((DIN, 64)),
            _wspec((1, 64)),
            _wspec((64, DH)),
            _wspec((1, DH)),
        ],
        out_specs=pl.BlockSpec((BT, DH), lambda i: (i, 0)),
        out_shape=jax.ShapeDtypeStruct((N, DH), jnp.float32),
    )(x, We1, be1, We2, be2)


def _dinv_of(deg_ref):
    d = deg_ref[...]
    dsum = d[0][:, 0:1] + d[1][:, 0:1] + 1.0
    return lax.rsqrt(dsum)


def _tc_pre1(deg, h0, Wg1):
    def body(deg_ref, h_ref, w_ref, o_ref):
        dinv = _dinv_of(deg_ref)
        o_ref[...] = dinv * (h_ref[...] @ w_ref[...])

    return pl.pallas_call(
        body,
        grid=(N // BT,),
        in_specs=[
            pl.BlockSpec((2, BT, DEGW), lambda i: (0, i, 0)),
            pl.BlockSpec((BT, DH), lambda i: (i, 0)),
            _wspec((DH, DH)),
        ],
        out_specs=pl.BlockSpec((BT, DH), lambda i: (i, 0)),
        out_shape=jax.ShapeDtypeStruct((NPAD, DH), jnp.float32),
    )(deg, h0, Wg1)


def _tc_mid(deg, A, g, b, Wnext):
    def body(deg_ref, a_ref, g_ref, b_ref, w_ref, o_ref):
        dinv = _dinv_of(deg_ref)
        a = a_ref[...]
        s = a[0] + a[1] + g_ref[...]
        h = jnp.maximum(dinv * s + b_ref[...], 0.0)
        o_ref[...] = dinv * (h @ w_ref[...])

    return pl.pallas_call(
        body,
        grid=(N // BT,),
        in_specs=[
            pl.BlockSpec((2, BT, DEGW), lambda i: (0, i, 0)),
            pl.BlockSpec((2, BT, DH), lambda i: (0, i, 0)),
            pl.BlockSpec((BT, DH), lambda i: (i, 0)),
            _wspec((1, DH)),
            _wspec((DH, DH)),
        ],
        out_specs=pl.BlockSpec((BT, DH), lambda i: (i, 0)),
        out_shape=jax.ShapeDtypeStruct((NPAD, DH), jnp.float32),
    )(deg, A, g, b, Wnext)


def _tc_final(deg, A, g, bg3, Wp1, bp1, Wp2, bp2, priors):
    def body(deg_ref, a_ref, g_ref, b3_ref, w1_ref, b1_ref, w2_ref, b2_ref,
             p_ref, o_ref):
        dinv = _dinv_of(deg_ref)
        a = a_ref[...]
        s = a[0] + a[1] + g_ref[...]
        h = jnp.maximum(dinv * s + b3_ref[...], 0.0)
        t = jnp.tanh(h @ w1_ref[...] + b1_ref[...])
        o_ref[...] = jnp.tanh(t @ w2_ref[...] + b2_ref[...]) + p_ref[...]

    return pl.pallas_call(
        body,
        grid=(N // BT,),
        in_specs=[
            pl.BlockSpec((2, BT, DEGW), lambda i: (0, i, 0)),
            pl.BlockSpec((2, BT, DH), lambda i: (0, i, 0)),
            pl.BlockSpec((BT, DH), lambda i: (i, 0)),
            _wspec((1, DH)),
            _wspec((DH, DH)),
            _wspec((1, DH)),
            _wspec((DH, 16)),
            _wspec((1, 16)),
            pl.BlockSpec((BT, 16), lambda i: (i, 0)),
        ],
        out_specs=pl.BlockSpec((BT, 16), lambda i: (i, 0)),
        out_shape=jax.ShapeDtypeStruct((N, 16), jnp.float32),
    )(deg, A, g, bg3, Wp1, bp1, Wp2, bp2, priors)


def kernel(x, edge_index, priors, We1, be1, We2, be2, Wg1, bg1, Wg2, bg2,
           Wg3, bg3, Wp1, bp1, Wp2, bp2):
    src3 = edge_index[0].reshape(NW, KB, CB)
    dst3 = edge_index[1].reshape(NW, KB, CB)
    z32 = jnp.zeros((NPAD, DH), jnp.float32)
    z8 = jnp.zeros((NPAD, DEGW), jnp.float32)
    ones_c = jnp.ones((CB, DEGW), jnp.float32)

    deg = _sc_deg(dst3, ones_c, z8)
    h0 = _tc_embed(x, We1, be1.reshape(1, -1), We2, be2.reshape(1, -1))

    g1 = _tc_pre1(deg, h0, Wg1)
    A1 = _sc_agg(g1, src3, dst3, z32)
    g2 = _tc_mid(deg, A1, g1, bg1.reshape(1, -1), Wg2)
    A2 = _sc_agg(g2, src3, dst3, z32)
    g3 = _tc_mid(deg, A2, g2, bg2.reshape(1, -1), Wg3)
    A3 = _sc_agg(g3, src3, dst3, z32)
    out = _tc_final(deg, A3, g3, bg3.reshape(1, -1), Wp1, bp1.reshape(1, -1),
                    Wp2, bp2.reshape(1, -1), priors)
    return out
